# Initial kernel scaffold; baseline (speedup 1.0000x reference)
import jax
import jax.numpy as jnp
from jax.experimental import pallas as pl


def kernel(c, edges, W):
    raise NotImplementedError("write your pallas kernel here")


# speedup vs baseline: 1667.6874x; 1667.6874x over previous
import functools

import jax
import jax.numpy as jnp
from jax.experimental import pallas as pl
from jax.experimental.pallas import tpu as pltpu
from jax.experimental.pallas import tpu_sc as plsc


def _matmul_body(c_ref, w_ref, p_ref):
    p_ref[...] = jnp.dot(c_ref[...], w_ref[...],
                         preferred_element_type=jnp.float32)


def _dot_body(a_ref, b_ref, o_ref):
    o_ref[...] = jnp.sum(a_ref[...] * b_ref[...], axis=1)[None, :]


def _sc_gather(table, idx2d, n_idx, C, window=128):
    mesh = plsc.VectorSubcoreMesh(core_axis_name="c", subcore_axis_name="s")

    @functools.partial(
        pl.kernel,
        out_type=jax.ShapeDtypeStruct((n_idx, C), jnp.float32),
        mesh=mesh,
    )
    def gk(t_hbm, i_hbm, o_hbm):
        def body(i_vmem, o_vmem):
            pltpu.sync_copy(t_hbm.at[i_vmem.at[0]], o_vmem)

        pltpu.emit_pipeline(
            body,
            grid=(n_idx // window,),
            in_specs=[pl.BlockSpec((1, window), index_map=lambda i: (0, i))],
            out_specs=[pl.BlockSpec((window, C), index_map=lambda i: (i, 0))],
            core_axis_name=("c", "s"),
            dimension_semantics=(pltpu.PARALLEL,),
        )(i_hbm, o_hbm)

    return gk(table, idx2d)


def kernel(c, edges, W):
    B, N, C = c.shape
    E = edges.shape[1]
    c0 = c[0]
    w0 = W[0]
    idx = edges[0].astype(jnp.int32)

    p = pl.pallas_call(
        _matmul_body,
        out_shape=jax.ShapeDtypeStruct((N, C), jnp.float32),
    )(c0, w0)

    table = jnp.concatenate([p, c0], axis=0)
    allidx = jnp.concatenate([idx[:, 0], idx[:, 1] + N]).reshape(1, 2 * E)
    g = _sc_gather(table, allidx, 2 * E, C)

    Eb = 3200
    nblk = E // Eb
    logits = pl.pallas_call(
        _dot_body,
        grid=(nblk,),
        in_specs=[
            pl.BlockSpec((Eb, C), lambda ii: (ii, 0)),
            pl.BlockSpec((Eb, C), lambda ii: (ii + nblk, 0)),
        ],
        out_specs=pl.BlockSpec((1, Eb), lambda ii: (0, ii)),
        out_shape=jax.ShapeDtypeStruct((1, E), jnp.float32),
    )(g, g)

    return logits


# speedup vs baseline: 1879.7816x; 1.1272x over previous
import functools

import jax
import jax.numpy as jnp
from jax.experimental import pallas as pl
from jax.experimental.pallas import tpu as pltpu
from jax.experimental.pallas import tpu_sc as plsc


def _matmul_body(c_ref, w_ref, p_ref):
    p_ref[...] = jnp.dot(c_ref[...], w_ref[...],
                         preferred_element_type=jnp.float32)


def _dot_body(a_ref, b_ref, o_ref):
    o_ref[...] = jnp.sum(a_ref[...] * b_ref[...], axis=1)[None, :]


def _sc_gather(table, idx2d, n_idx, C, window=128):
    mesh = plsc.VectorSubcoreMesh(core_axis_name="c", subcore_axis_name="s")

    @functools.partial(
        pl.kernel,
        out_type=jax.ShapeDtypeStruct((n_idx, C), jnp.float32),
        mesh=mesh,
    )
    def gk(t_hbm, i_hbm, o_hbm):
        def body(i_vmem, o_vmem):
            pltpu.sync_copy(t_hbm.at[i_vmem.at[0]], o_vmem)

        pltpu.emit_pipeline(
            body,
            grid=(n_idx // window,),
            in_specs=[pl.BlockSpec((1, window), index_map=lambda i: (0, i))],
            out_specs=[pl.BlockSpec((window, C), index_map=lambda i: (i, 0))],
            core_axis_name=("c", "s"),
            dimension_semantics=(pltpu.PARALLEL,),
        )(i_hbm, o_hbm)

    return gk(table, idx2d)


def kernel(c, edges, W):
    B, N, C = c.shape
    E = edges.shape[1]
    c0 = c[0]
    w0 = W[0]
    idx = edges[0].astype(jnp.int32)

    p = pl.pallas_call(
        _matmul_body,
        out_shape=jax.ShapeDtypeStruct((N, C), jnp.float32),
    )(c0, w0)

    table = jnp.concatenate([p, c0], axis=0)
    allidx = jnp.concatenate([idx[:, 0], idx[:, 1] + N]).reshape(1, 2 * E)
    g = _sc_gather(table, allidx, 2 * E, C, window=256)

    Eb = 3200
    nblk = E // Eb
    logits = pl.pallas_call(
        _dot_body,
        grid=(nblk,),
        in_specs=[
            pl.BlockSpec((Eb, C), lambda ii: (ii, 0)),
            pl.BlockSpec((Eb, C), lambda ii: (ii + nblk, 0)),
        ],
        out_specs=pl.BlockSpec((1, Eb), lambda ii: (0, ii)),
        out_shape=jax.ShapeDtypeStruct((1, E), jnp.float32),
    )(g, g)

    return logits


# speedup vs baseline: 1962.9485x; 1.0442x over previous
import functools

import jax
import jax.numpy as jnp
from jax.experimental import pallas as pl
from jax.experimental.pallas import tpu as pltpu
from jax.experimental.pallas import tpu_sc as plsc


def _matmul_body(c_ref, w_ref, p_ref):
    p_ref[...] = jnp.dot(c_ref[...], w_ref[...],
                         preferred_element_type=jnp.float32)


def _dot_body(a_ref, b_ref, o_ref):
    a = a_ref[...].astype(jnp.float32)
    b = b_ref[...].astype(jnp.float32)
    o_ref[...] = jnp.sum(a * b, axis=1)[None, :]


def _sc_gather(table, idx2d, n_idx, C, window=128):
    mesh = plsc.VectorSubcoreMesh(core_axis_name="c", subcore_axis_name="s")

    @functools.partial(
        pl.kernel,
        out_type=jax.ShapeDtypeStruct((n_idx, C), table.dtype),
        mesh=mesh,
    )
    def gk(t_hbm, i_hbm, o_hbm):
        def body(i_vmem, o_vmem):
            pltpu.sync_copy(t_hbm.at[i_vmem.at[0]], o_vmem)

        pltpu.emit_pipeline(
            body,
            grid=(n_idx // window,),
            in_specs=[pl.BlockSpec((1, window), index_map=lambda i: (0, i))],
            out_specs=[pl.BlockSpec((window, C), index_map=lambda i: (i, 0))],
            core_axis_name=("c", "s"),
            dimension_semantics=(pltpu.PARALLEL,),
        )(i_hbm, o_hbm)

    return gk(table, idx2d)


def kernel(c, edges, W):
    B, N, C = c.shape
    E = edges.shape[1]
    c0 = c[0]
    w0 = W[0]
    idx = edges[0].astype(jnp.int32)

    p = pl.pallas_call(
        _matmul_body,
        out_shape=jax.ShapeDtypeStruct((N, C), jnp.float32),
    )(c0, w0)

    table = jnp.concatenate([p, c0], axis=0)
    i_all = idx[:, 0]
    j_all = idx[:, 1] + N

    S = 4
    Es = E // S
    Eb = 3200
    nblk = Es // Eb
    parts = []
    for s in range(S):
        sl = slice(s * Es, (s + 1) * Es)
        allidx = jnp.concatenate([i_all[sl], j_all[sl]]).reshape(1, 2 * Es)
        g = _sc_gather(table, allidx, 2 * Es, C, window=256)
        part = pl.pallas_call(
            _dot_body,
            grid=(nblk,),
            in_specs=[
                pl.BlockSpec((Eb, C), lambda ii: (ii, 0)),
                pl.BlockSpec((Eb, C), lambda ii: (ii + nblk, 0)),
            ],
            out_specs=pl.BlockSpec((1, Eb), lambda ii: (0, ii)),
            out_shape=jax.ShapeDtypeStruct((1, Es), jnp.float32),
        )(g, g)
        parts.append(part)

    return jnp.concatenate(parts, axis=1)
